# baseline (speedup 1.0000x reference)
import jax
import jax.numpy as jnp
from jax import lax
from jax.experimental import pallas as pl
from jax.experimental.pallas import tpu as pltpu
from jax.experimental.pallas import tpu_sc as plsc

B = 4096
L = 200
E = 32
NC = 2
NS = 16
NW = NC * NS
LT = L // 8
NH = 2
LPH = 4
RH = LPH * 128
UNITS = LT * NH
NBUF = 2


def _sc_body(xt_hbm, w_hbm, out_hbm, idx_v, rows_v, outt_0, outt_1, gsem,
             osem):
  outts = (outt_0, outt_1)
  wid = lax.axis_index("s") * NC + lax.axis_index("c")

  iota = lax.iota(jnp.int32, 16)
  e_lo = iota
  e_hi = iota + 16

  def start_gather(u, k):
    lt = u // NH
    h = u % NH
    pltpu.sync_copy(xt_hbm.at[wid].at[lt].at[pl.ds(h * LPH, LPH)],
                    idx_v.at[k])
    for li in range(LPH):
      pltpu.async_copy(
          w_hbm.at[idx_v.at[k].at[li]],
          rows_v.at[k].at[pl.ds(li * 128, 128)],
          gsem.at[k],
      )

  def wait_gather(k):
    for li in range(LPH):
      pltpu.make_async_copy(
          w_hbm.at[idx_v.at[k].at[li]],
          rows_v.at[k].at[pl.ds(li * 128, 128)],
          gsem.at[k],
      ).wait()

  def transpose(k):
    rows = rows_v.at[k]
    outt = outts[k]

    def step(i, _):
      for uu in range(4):
        r = i * 4 + uu
        rv = jnp.full((16,), r, jnp.int32)
        plsc.store_scatter(outt, [e_lo, rv], rows[r, 0:16])
        plsc.store_scatter(outt, [e_hi, rv], rows[r, 16:32])
      return 0

    lax.fori_loop(0, RH // 4, step, 0)

  def out_dst(u):
    lt = u // NH
    h = u % NH
    return out_hbm.at[:, lt, wid, h]

  for k in range(NBUF):
    start_gather(k, k)

  def outer(g, _):
    for k in range(NBUF):
      u = g * NBUF + k
      wait_gather(k)

      @pl.when(g > 0)
      def _():
        pltpu.make_async_copy(outts[k], out_dst(u), osem.at[k]).wait()

      transpose(k)
      pltpu.async_copy(outts[k], out_dst(u), osem.at[k])

      @pl.when(u + NBUF < UNITS)
      def _():
        start_gather(u + NBUF, k)

    return 0

  lax.fori_loop(0, UNITS // NBUF, outer, 0)

  for k in range(NBUF):
    pltpu.make_async_copy(outts[k], out_hbm.at[:, 0, wid, 0],
                          osem.at[k]).wait()


@jax.jit
def kernel(x, W):
  xt = x.astype(jnp.int32).reshape(NW, 128, LT, 8).transpose(0, 2, 3, 1)
  run = pl.kernel(
      _sc_body,
      out_type=jax.ShapeDtypeStruct((E, LT, NW, NH, RH), jnp.float32),
      mesh=plsc.VectorSubcoreMesh(core_axis_name="c", subcore_axis_name="s"),
      compiler_params=pltpu.CompilerParams(
          use_tc_tiling_on_sc=False, needs_layout_passes=False
      ),
      scratch_types=[
          pltpu.VMEM((NBUF, LPH, 128), jnp.int32),
          pltpu.VMEM((NBUF, RH, E), jnp.float32),
          pltpu.VMEM((E, RH), jnp.float32),
          pltpu.VMEM((E, RH), jnp.float32),
          pltpu.SemaphoreType.DMA((NBUF,)),
          pltpu.SemaphoreType.DMA((NBUF,)),
      ],
  )
  out6 = run(xt, W)
  out = out6.reshape(E, LT, NW, 8, 128).transpose(2, 4, 0, 1, 3)
  return out.reshape(B, E, L)


# speedup vs baseline: 1.0514x; 1.0514x over previous
import jax
import jax.numpy as jnp
from jax import lax
from jax.experimental import pallas as pl
from jax.experimental.pallas import tpu as pltpu
from jax.experimental.pallas import tpu_sc as plsc

B = 4096
L = 200
E = 32
NC = 2
NS = 16
NW = NC * NS
LT = L // 8
NH = 2
LPH = 4
RH = LPH * 128
UNITS = LT * NH
NBUF = 2


def _sc_body(xt_hbm, w_hbm, out_hbm, idx_v, rows_v, outt_0, outt_1, gsem,
             osem):
  outts = (outt_0, outt_1)
  wid = lax.axis_index("s") * NC + lax.axis_index("c")

  iota = lax.iota(jnp.int32, 16)
  e_lo = iota
  e_hi = iota + 16

  pltpu.sync_copy(xt_hbm.at[wid], idx_v)

  def start_gather(u, k):
    for li in range(LPH):
      pltpu.async_copy(
          w_hbm.at[idx_v.at[u * LPH + li]],
          rows_v.at[k].at[pl.ds(li * 128, 128)],
          gsem.at[k],
      )

  def wait_gather(u, k):
    for li in range(LPH):
      pltpu.make_async_copy(
          w_hbm.at[idx_v.at[u * LPH + li]],
          rows_v.at[k].at[pl.ds(li * 128, 128)],
          gsem.at[k],
      ).wait()

  def transpose(k):
    rows = rows_v.at[k]
    outt = outts[k]

    def step(i, _):
      for uu in range(4):
        r = i * 4 + uu
        rv = jnp.full((16,), r, jnp.int32)
        plsc.store_scatter(outt, [e_lo, rv], rows[r, 0:16])
        plsc.store_scatter(outt, [e_hi, rv], rows[r, 16:32])
      return 0

    lax.fori_loop(0, RH // 4, step, 0)

  def out_dst(u):
    lt = u // NH
    h = u % NH
    return out_hbm.at[:, lt, wid, h]

  for k in range(NBUF):
    start_gather(k, k)

  def outer(g, _):
    for k in range(NBUF):
      u = g * NBUF + k
      wait_gather(u, k)

      @pl.when(g > 0)
      def _():
        pltpu.make_async_copy(outts[k], out_dst(u), osem.at[k]).wait()

      transpose(k)
      pltpu.async_copy(outts[k], out_dst(u), osem.at[k])

      @pl.when(u + NBUF < UNITS)
      def _():
        start_gather(u + NBUF, k)

    return 0

  lax.fori_loop(0, UNITS // NBUF, outer, 0)

  for k in range(NBUF):
    pltpu.make_async_copy(outts[k], out_hbm.at[:, 0, wid, 0],
                          osem.at[k]).wait()


@jax.jit
def kernel(x, W):
  xt = x.astype(jnp.int32).reshape(NW, 128, LT, 8).transpose(0, 2, 3, 1)
  xt = xt.reshape(NW, LT * 8, 128)
  run = pl.kernel(
      _sc_body,
      out_type=jax.ShapeDtypeStruct((E, LT, NW, NH, RH), jnp.float32),
      mesh=plsc.VectorSubcoreMesh(core_axis_name="c", subcore_axis_name="s"),
      compiler_params=pltpu.CompilerParams(
          use_tc_tiling_on_sc=False, needs_layout_passes=False
      ),
      scratch_types=[
          pltpu.VMEM((LT * 8, 128), jnp.int32),
          pltpu.VMEM((NBUF, RH, E), jnp.float32),
          pltpu.VMEM((E, RH), jnp.float32),
          pltpu.VMEM((E, RH), jnp.float32),
          pltpu.SemaphoreType.DMA((NBUF,)),
          pltpu.SemaphoreType.DMA((NBUF,)),
      ],
  )
  out6 = run(xt, W)
  out = out6.reshape(E, LT, NW, 8, 128).transpose(2, 4, 0, 1, 3)
  return out.reshape(B, E, L)


# speedup vs baseline: 1.4123x; 1.3433x over previous
import jax
import jax.numpy as jnp
from jax import lax
from jax.experimental import pallas as pl
from jax.experimental.pallas import tpu as pltpu
from jax.experimental.pallas import tpu_sc as plsc

B = 4096
L = 200
E = 32
NC = 2
NS = 16
NW = NC * NS
LT = L // 8
NH = 2
LPH = 4
RH = LPH * 128
UNITS = LT * NH
NBUF = 2


def _sc_body(xt_hbm, w_hbm, out_hbm, idx_v, rows_v, outt_0, outt_1, gsem,
             osem):
  outts = (outt_0, outt_1)
  wid = lax.axis_index("s") * NC + lax.axis_index("c")

  iota = lax.iota(jnp.int32, 16)
  e_lo = iota
  e_hi = iota + 16

  pltpu.sync_copy(xt_hbm.at[wid], idx_v)

  def start_gather(u, k):
    for li in range(LPH):
      pltpu.async_copy(
          w_hbm.at[idx_v.at[u * LPH + li]],
          rows_v.at[k].at[pl.ds(li * 128, 128)],
          gsem.at[k],
      )

  def wait_gather(u, k):
    for li in range(LPH):
      pltpu.make_async_copy(
          w_hbm.at[idx_v.at[u * LPH + li]],
          rows_v.at[k].at[pl.ds(li * 128, 128)],
          gsem.at[k],
      ).wait()

  def transpose(k):
    rows = rows_v.at[k]
    outt = outts[k]

    @plsc.parallel_loop(0, RH, unroll=8, carry=jnp.zeros((16,), jnp.int32))
    def _(r, rv):
      plsc.store_scatter(outt, [e_lo, rv], rows[r, 0:16])
      plsc.store_scatter(outt, [e_hi, rv], rows[r, 16:32])
      return rv + 1

  def out_dst(u):
    lt = u // NH
    h = u % NH
    return out_hbm.at[:, lt, wid, h]

  for k in range(NBUF):
    start_gather(k, k)

  def outer(g, _):
    for k in range(NBUF):
      u = g * NBUF + k
      wait_gather(u, k)

      @pl.when(g > 0)
      def _():
        pltpu.make_async_copy(outts[k], out_dst(u), osem.at[k]).wait()

      transpose(k)
      pltpu.async_copy(outts[k], out_dst(u), osem.at[k])

      @pl.when(u + NBUF < UNITS)
      def _():
        start_gather(u + NBUF, k)

    return 0

  lax.fori_loop(0, UNITS // NBUF, outer, 0)

  for k in range(NBUF):
    pltpu.make_async_copy(outts[k], out_hbm.at[:, 0, wid, 0],
                          osem.at[k]).wait()


@jax.jit
def kernel(x, W):
  xt = x.astype(jnp.int32).reshape(NW, 128, LT, 8).transpose(0, 2, 3, 1)
  xt = xt.reshape(NW, LT * 8, 128)
  run = pl.kernel(
      _sc_body,
      out_type=jax.ShapeDtypeStruct((E, LT, NW, NH, RH), jnp.float32),
      mesh=plsc.VectorSubcoreMesh(core_axis_name="c", subcore_axis_name="s"),
      compiler_params=pltpu.CompilerParams(
          use_tc_tiling_on_sc=False, needs_layout_passes=False
      ),
      scratch_types=[
          pltpu.VMEM((LT * 8, 128), jnp.int32),
          pltpu.VMEM((NBUF, RH, E), jnp.float32),
          pltpu.VMEM((E, RH), jnp.float32),
          pltpu.VMEM((E, RH), jnp.float32),
          pltpu.SemaphoreType.DMA((NBUF,)),
          pltpu.SemaphoreType.DMA((NBUF,)),
      ],
  )
  out6 = run(xt, W)
  out = out6.reshape(E, LT, NW, 8, 128).transpose(2, 4, 0, 1, 3)
  return out.reshape(B, E, L)


# speedup vs baseline: 1.4147x; 1.0017x over previous
import jax
import jax.numpy as jnp
from jax import lax
from jax.experimental import pallas as pl
from jax.experimental.pallas import tpu as pltpu
from jax.experimental.pallas import tpu_sc as plsc

B = 4096
L = 200
E = 32
NC = 2
NS = 16
NW = NC * NS
LT = L // 8
LPH = 2
NH = 8 // LPH
RH = LPH * 128
UNITS = LT * NH
NBUF = 4


def _sc_body(xt_hbm, w_hbm, out_hbm, idx_v, rows_0, rows_1, rows_2, rows_3,
             outt_0, outt_1, outt_2, outt_3, gsem, osem):
  rows_v = (rows_0, rows_1, rows_2, rows_3)
  outts = (outt_0, outt_1, outt_2, outt_3)
  wid = lax.axis_index("s") * NC + lax.axis_index("c")

  iota = lax.iota(jnp.int32, 16)
  e_lo = iota
  e_hi = iota + 16

  pltpu.sync_copy(xt_hbm.at[wid], idx_v)

  def start_gather(u, k):
    for li in range(LPH):
      pltpu.async_copy(
          w_hbm.at[idx_v.at[u * LPH + li]],
          rows_v[k].at[pl.ds(li * 128, 128)],
          gsem.at[k],
      )

  def wait_gather(u, k):
    for li in range(LPH):
      pltpu.make_async_copy(
          w_hbm.at[idx_v.at[u * LPH + li]],
          rows_v[k].at[pl.ds(li * 128, 128)],
          gsem.at[k],
      ).wait()

  def transpose(k):
    rows = rows_v[k]
    outt = outts[k]

    @plsc.parallel_loop(0, RH, unroll=8, carry=jnp.zeros((16,), jnp.int32))
    def _(r, rv):
      plsc.store_scatter(outt, [e_lo, rv], rows[r, 0:16])
      plsc.store_scatter(outt, [e_hi, rv], rows[r, 16:32])
      return rv + 1

  def out_dst(u):
    lt = u // NH
    h = u % NH
    return out_hbm.at[:, lt, wid, h]

  for k in range(NBUF):
    start_gather(k, k)

  def outer(g, _):
    for k in range(NBUF):
      u = g * NBUF + k
      wait_gather(u, k)

      @pl.when(g > 0)
      def _():
        pltpu.make_async_copy(outts[k], out_dst(u), osem.at[k]).wait()

      transpose(k)
      pltpu.async_copy(outts[k], out_dst(u), osem.at[k])

      @pl.when(u + NBUF < UNITS)
      def _():
        start_gather(u + NBUF, k)

    return 0

  lax.fori_loop(0, UNITS // NBUF, outer, 0)

  for k in range(NBUF):
    pltpu.make_async_copy(outts[k], out_hbm.at[:, 0, wid, 0],
                          osem.at[k]).wait()


@jax.jit
def kernel(x, W):
  xt = x.astype(jnp.int32).reshape(NW, 128, LT, 8).transpose(0, 2, 3, 1)
  xt = xt.reshape(NW, LT * 8, 128)
  run = pl.kernel(
      _sc_body,
      out_type=jax.ShapeDtypeStruct((E, LT, NW, NH, RH), jnp.float32),
      mesh=plsc.VectorSubcoreMesh(core_axis_name="c", subcore_axis_name="s"),
      compiler_params=pltpu.CompilerParams(
          use_tc_tiling_on_sc=False, needs_layout_passes=False
      ),
      scratch_types=(
          [pltpu.VMEM((LT * 8, 128), jnp.int32)]
          + [pltpu.VMEM((RH, E), jnp.float32) for _ in range(NBUF)]
          + [pltpu.VMEM((E, RH), jnp.float32) for _ in range(NBUF)]
          + [pltpu.SemaphoreType.DMA((NBUF,)),
             pltpu.SemaphoreType.DMA((NBUF,))]
      ),
  )
  out6 = run(xt, W)
  out = out6.reshape(E, LT, NW, 8, 128).transpose(2, 4, 0, 1, 3)
  return out.reshape(B, E, L)
